# Initial kernel scaffold; baseline (speedup 1.0000x reference)
import jax
import jax.numpy as jnp
from jax.experimental import pallas as pl


def kernel(x, edge_index, batch, W1, b1, W2, b2):
    raise NotImplementedError("write your pallas kernel here")


# speedup vs baseline: 13.1602x; 13.1602x over previous
import functools

import jax
import jax.numpy as jnp
from jax import lax
from jax.experimental import pallas as pl
from jax.experimental.pallas import tpu as pltpu
from jax.experimental.pallas import tpu_sc as plsc

NN = 10000
NE = 320000
D = 128
NG = 64

NC = 2
NS = 16
CH = 80
EPT = NE // (NC * NS)
NCH = EPT // CH
NNP = 10240
ROWS_PT = NNP // NS
DRAIN = 128
DEGP = 10240
DEG_PT = DEGP // NS


@functools.lru_cache(maxsize=None)
def _build_deg_kernel():
    mesh = plsc.VectorSubcoreMesh(core_axis_name="c", subcore_axis_name="s")

    @functools.partial(
        pl.kernel,
        mesh=mesh,
        out_type=jax.ShapeDtypeStruct((NC * DEGP,), jnp.float32),
        scratch_types=[
            pltpu.VMEM((CH,), jnp.int32),
            pltpu.VMEM((CH,), jnp.float32),
            pltpu.VMEM((DEG_PT,), jnp.float32),
            pltpu.VMEM_SHARED((DEGP,), jnp.float32),
        ],
    )
    def _deg_kernel(dst_hbm, ones_hbm, zeros_hbm, out_hbm, dst_v, ones_v, buf_v, acc_sh):
        c = lax.axis_index("c")
        s = lax.axis_index("s")
        pltpu.sync_copy(zeros_hbm, buf_v)
        pltpu.sync_copy(buf_v, acc_sh.at[pl.ds(s * DEG_PT, DEG_PT)])
        pltpu.sync_copy(ones_hbm, ones_v)
        plsc.subcore_barrier()

        ebase = c * (NS * EPT) + s * EPT

        def body(i, carry):
            base = ebase + i * CH
            pltpu.sync_copy(dst_hbm.at[pl.ds(base, CH)], dst_v)
            pltpu.sync_copy(ones_v, acc_sh.at[dst_v], add=True)
            return carry

        lax.fori_loop(0, NCH, body, 0)
        plsc.subcore_barrier()

        pltpu.sync_copy(acc_sh.at[pl.ds(s * DEG_PT, DEG_PT)], buf_v)
        pltpu.sync_copy(buf_v, out_hbm.at[pl.ds(c * DEGP + s * DEG_PT, DEG_PT)])

    return _deg_kernel


@functools.lru_cache(maxsize=None)
def _build_agg_kernel():
    mesh = plsc.VectorSubcoreMesh(core_axis_name="c", subcore_axis_name="s")

    @functools.partial(
        pl.kernel,
        mesh=mesh,
        out_type=jax.ShapeDtypeStruct((NC * NNP, D), jnp.float32),
        scratch_types=[
            pltpu.VMEM((CH,), jnp.int32),
            pltpu.VMEM((CH,), jnp.int32),
            pltpu.VMEM((CH, D), jnp.float32),
            pltpu.VMEM((DRAIN, D), jnp.float32),
            pltpu.VMEM_SHARED((NNP, D), jnp.float32),
            pltpu.SemaphoreType.DMA,
        ],
    )
    def _agg_kernel(g_hbm, src_hbm, dst_hbm, zeros_hbm, out_hbm,
                    src_v, dst_v, rows_v, buf_v, acc_sh, sem):
        c = lax.axis_index("c")
        s = lax.axis_index("s")
        pltpu.sync_copy(zeros_hbm, buf_v)
        for k in range(ROWS_PT // DRAIN):
            pltpu.sync_copy(buf_v, acc_sh.at[pl.ds(s * ROWS_PT + k * DRAIN, DRAIN)])
        plsc.subcore_barrier()

        ebase = c * (NS * EPT) + s * EPT

        def body(i, carry):
            base = ebase + i * CH
            pltpu.sync_copy(src_hbm.at[pl.ds(base, CH)], src_v)
            pltpu.sync_copy(dst_hbm.at[pl.ds(base, CH)], dst_v)
            pltpu.async_copy(g_hbm.at[src_v], rows_v, sem).wait()
            pltpu.sync_copy(rows_v, acc_sh.at[dst_v], add=True)
            return carry

        lax.fori_loop(0, NCH, body, 0)
        plsc.subcore_barrier()

        for k in range(ROWS_PT // DRAIN):
            r = s * ROWS_PT + k * DRAIN
            pltpu.sync_copy(acc_sh.at[pl.ds(r, DRAIN)], buf_v)
            pltpu.sync_copy(buf_v, out_hbm.at[pl.ds(c * NNP + r, DRAIN)])

    return _agg_kernel


def _dense1_body(x_ref, w_ref, d0_ref, d1_ref, g_ref):
    dinv = lax.rsqrt(d0_ref[...] + d1_ref[...] + 1.0)
    h = jnp.dot(x_ref[...], w_ref[...], preferred_element_type=jnp.float32)
    g_ref[...] = h * dinv


def _dense2_body(agg_ref, g1_ref, d0_ref, d1_ref, w_ref, b_ref, g2_ref):
    dinv = lax.rsqrt(d0_ref[...] + d1_ref[...] + 1.0)
    a = agg_ref[pl.ds(0, NN), :] + agg_ref[pl.ds(NNP, NN), :] + g1_ref[...]
    out1 = jnp.maximum(a * dinv + b_ref[...], 0.0)
    h2 = jnp.dot(out1, w_ref[...], preferred_element_type=jnp.float32)
    g2_ref[...] = h2 * dinv


def _final_body(agg_ref, g2_ref, d0_ref, d1_ref, b_ref, batch_ref, out_ref):
    dinv = lax.rsqrt(d0_ref[...] + d1_ref[...] + 1.0)
    a = agg_ref[pl.ds(0, NN), :] + agg_ref[pl.ds(NNP, NN), :] + g2_ref[...]
    out2 = jnp.maximum(a * dinv + b_ref[...], 0.0)
    gids = lax.broadcasted_iota(jnp.int32, (NG, NN), 0)
    p = (gids == batch_ref[...]).astype(jnp.float32)
    sums = jnp.dot(p, out2, preferred_element_type=jnp.float32)
    counts = jnp.sum(p, axis=1, keepdims=True)
    out_ref[...] = sums / jnp.maximum(counts, 1.0)


_dense1 = pl.pallas_call(
    _dense1_body, out_shape=jax.ShapeDtypeStruct((NN, D), jnp.float32))
_dense2 = pl.pallas_call(
    _dense2_body, out_shape=jax.ShapeDtypeStruct((NN, D), jnp.float32))
_final = pl.pallas_call(
    _final_body, out_shape=jax.ShapeDtypeStruct((NG, D), jnp.float32))


def kernel(x, edge_index, batch, W1, b1, W2, b2):
    src = edge_index[0].astype(jnp.int32)
    dst = edge_index[1].astype(jnp.int32)
    batch2d = batch.astype(jnp.int32).reshape(1, NN)
    ones_ch = jnp.ones((CH,), jnp.float32)
    zeros_deg = jnp.zeros((DEG_PT,), jnp.float32)
    zeros_agg = jnp.zeros((DRAIN, D), jnp.float32)
    b1r = b1.reshape(1, D)
    b2r = b2.reshape(1, D)

    deg_k = _build_deg_kernel()
    agg_k = _build_agg_kernel()

    deg2 = deg_k(dst, ones_ch, zeros_deg)
    d0 = deg2[:NN].reshape(NN, 1)
    d1 = deg2[DEGP:DEGP + NN].reshape(NN, 1)

    g1 = _dense1(x, W1, d0, d1)
    a1 = agg_k(g1, src, dst, zeros_agg)
    g2 = _dense2(a1, g1, d0, d1, W2, b1r)
    a2 = agg_k(g2, src, dst, zeros_agg)
    return _final(a2, g2, d0, d1, b2r, batch2d)


# speedup vs baseline: 25.7225x; 1.9546x over previous
import functools

import jax
import jax.numpy as jnp
from jax import lax
from jax.experimental import pallas as pl
from jax.experimental.pallas import tpu as pltpu
from jax.experimental.pallas import tpu_sc as plsc

NN = 10000
NE = 320000
D = 128
NG = 64

NC = 2
NS = 16
CH = 80
EPT = NE // (NC * NS)
NCH = EPT // CH
NNP = 10240
ROWS_PT = NNP // NS
DRAIN = CH
DEGP = 10240
DEG_PT = DEGP // NS


@functools.lru_cache(maxsize=None)
def _build_deg_kernel():
    mesh = plsc.VectorSubcoreMesh(core_axis_name="c", subcore_axis_name="s")

    @functools.partial(
        pl.kernel,
        mesh=mesh,
        out_type=jax.ShapeDtypeStruct((NC * DEGP,), jnp.float32),
        scratch_types=[
            pltpu.VMEM((NCH, CH), jnp.int32),
            pltpu.VMEM((CH,), jnp.float32),
            pltpu.VMEM((DEG_PT,), jnp.float32),
            pltpu.VMEM_SHARED((DEGP,), jnp.float32),
        ],
    )
    def _deg_kernel(dst_hbm, ones_hbm, zeros_hbm, out_hbm, dst_v, ones_v, buf_v, acc_sh):
        c = lax.axis_index("c")
        s = lax.axis_index("s")
        wid = c * NS + s
        pltpu.sync_copy(zeros_hbm, buf_v)
        pltpu.sync_copy(buf_v, acc_sh.at[pl.ds(s * DEG_PT, DEG_PT)])
        pltpu.sync_copy(ones_hbm, ones_v)
        pltpu.sync_copy(dst_hbm.at[wid], dst_v)
        plsc.subcore_barrier()

        def body(i, carry):
            pltpu.sync_copy(ones_v, acc_sh.at[dst_v.at[i]], add=True)
            return carry

        lax.fori_loop(0, NCH, body, 0)
        plsc.subcore_barrier()

        pltpu.sync_copy(acc_sh.at[pl.ds(s * DEG_PT, DEG_PT)], buf_v)
        pltpu.sync_copy(buf_v, out_hbm.at[pl.ds(c * DEGP + s * DEG_PT, DEG_PT)])

    return _deg_kernel


@functools.lru_cache(maxsize=None)
def _build_agg_kernel():
    mesh = plsc.VectorSubcoreMesh(core_axis_name="c", subcore_axis_name="s")

    @functools.partial(
        pl.kernel,
        mesh=mesh,
        out_type=jax.ShapeDtypeStruct((NC * NNP, D), jnp.float32),
        scratch_types=[
            pltpu.VMEM((CH,), jnp.int32),
            pltpu.VMEM((CH,), jnp.int32),
            pltpu.VMEM((CH,), jnp.int32),
            pltpu.VMEM((CH,), jnp.int32),
            pltpu.VMEM((CH, D), jnp.float32),
            pltpu.VMEM((CH, D), jnp.float32),
            pltpu.VMEM_SHARED((NNP, D), jnp.float32),
            pltpu.SemaphoreType.DMA,
            pltpu.SemaphoreType.DMA,
            pltpu.SemaphoreType.DMA,
            pltpu.SemaphoreType.DMA,
        ],
    )
    def _agg_kernel(g_hbm, src_hbm, dst_hbm, zeros_hbm, out_hbm,
                    s0_v, d0_v, s1_v, d1_v, r0_v, r1_v, acc_sh,
                    isem0, isem1, gsem0, gsem1):
        c = lax.axis_index("c")
        s = lax.axis_index("s")
        eb = (c * NS + s) * EPT

        def idx_start(ch, sv, dv, sem):
            pltpu.async_copy(src_hbm.at[pl.ds(eb + ch * CH, CH)], sv, sem)
            pltpu.async_copy(dst_hbm.at[pl.ds(eb + ch * CH, CH)], dv, sem)

        def idx_wait(ch, sv, dv, sem):
            pltpu.make_async_copy(src_hbm.at[pl.ds(eb + ch * CH, CH)], sv, sem).wait()
            pltpu.make_async_copy(dst_hbm.at[pl.ds(eb + ch * CH, CH)], dv, sem).wait()

        def gather_start(sv, rv, sem):
            pltpu.async_copy(g_hbm.at[sv], rv, sem)

        def gather_wait(sv, rv, sem):
            pltpu.make_async_copy(g_hbm.at[sv], rv, sem).wait()

        pltpu.sync_copy(zeros_hbm, r0_v)
        for k in range(ROWS_PT // DRAIN):
            pltpu.sync_copy(r0_v, acc_sh.at[pl.ds(s * ROWS_PT + k * DRAIN, DRAIN)])
        plsc.subcore_barrier()

        idx_start(0, s0_v, d0_v, isem0)
        idx_start(1, s1_v, d1_v, isem1)
        idx_wait(0, s0_v, d0_v, isem0)
        gather_start(s0_v, r0_v, gsem0)

        def body(i, carry):
            a = 2 * i
            idx_wait(a + 1, s1_v, d1_v, isem1)
            gather_start(s1_v, r1_v, gsem1)
            gather_wait(s0_v, r0_v, gsem0)
            pltpu.sync_copy(r0_v, acc_sh.at[d0_v], add=True)
            ne = jnp.minimum(a + 2, NCH - 1)
            no = jnp.minimum(a + 3, NCH - 1)
            idx_start(ne, s0_v, d0_v, isem0)
            gather_wait(s1_v, r1_v, gsem1)
            pltpu.sync_copy(r1_v, acc_sh.at[d1_v], add=True)
            idx_start(no, s1_v, d1_v, isem1)
            idx_wait(ne, s0_v, d0_v, isem0)
            gather_start(s0_v, r0_v, gsem0)
            return carry

        lax.fori_loop(0, (NCH - 1) // 2, body, 0)
        gather_wait(s0_v, r0_v, gsem0)
        pltpu.sync_copy(r0_v, acc_sh.at[d0_v], add=True)
        idx_wait(NCH - 1, s1_v, d1_v, isem1)
        plsc.subcore_barrier()

        for k in range(ROWS_PT // DRAIN):
            r = s * ROWS_PT + k * DRAIN
            pltpu.sync_copy(acc_sh.at[pl.ds(r, DRAIN)], r0_v)
            pltpu.sync_copy(r0_v, out_hbm.at[pl.ds(c * NNP + r, DRAIN)])

    return _agg_kernel


def _dense1_body(x_ref, w_ref, d0_ref, d1_ref, g_ref):
    dinv = lax.rsqrt(d0_ref[...] + d1_ref[...] + 1.0)
    h = jnp.dot(x_ref[...], w_ref[...], preferred_element_type=jnp.float32)
    g_ref[...] = h * dinv


def _dense2_body(agg_ref, g1_ref, d0_ref, d1_ref, w_ref, b_ref, g2_ref):
    dinv = lax.rsqrt(d0_ref[...] + d1_ref[...] + 1.0)
    a = agg_ref[pl.ds(0, NN), :] + agg_ref[pl.ds(NNP, NN), :] + g1_ref[...]
    out1 = jnp.maximum(a * dinv + b_ref[...], 0.0)
    h2 = jnp.dot(out1, w_ref[...], preferred_element_type=jnp.float32)
    g2_ref[...] = h2 * dinv


def _final_body(agg_ref, g2_ref, d0_ref, d1_ref, b_ref, batch_ref, out_ref):
    dinv = lax.rsqrt(d0_ref[...] + d1_ref[...] + 1.0)
    a = agg_ref[pl.ds(0, NN), :] + agg_ref[pl.ds(NNP, NN), :] + g2_ref[...]
    out2 = jnp.maximum(a * dinv + b_ref[...], 0.0)
    gids = lax.broadcasted_iota(jnp.int32, (NG, NN), 0)
    p = (gids == batch_ref[...]).astype(jnp.float32)
    sums = jnp.dot(p, out2, preferred_element_type=jnp.float32)
    counts = jnp.sum(p, axis=1, keepdims=True)
    out_ref[...] = sums / jnp.maximum(counts, 1.0)


_dense1 = pl.pallas_call(
    _dense1_body, out_shape=jax.ShapeDtypeStruct((NN, D), jnp.float32))
_dense2 = pl.pallas_call(
    _dense2_body, out_shape=jax.ShapeDtypeStruct((NN, D), jnp.float32))
_final = pl.pallas_call(
    _final_body, out_shape=jax.ShapeDtypeStruct((NG, D), jnp.float32))


def kernel(x, edge_index, batch, W1, b1, W2, b2):
    src = edge_index[0].astype(jnp.int32)
    dst = edge_index[1].astype(jnp.int32)
    dst3 = dst.reshape(NC * NS, NCH, CH)
    batch2d = batch.astype(jnp.int32).reshape(1, NN)
    ones_ch = jnp.ones((CH,), jnp.float32)
    zeros_deg = jnp.zeros((DEG_PT,), jnp.float32)
    zeros_agg = jnp.zeros((CH, D), jnp.float32)
    b1r = b1.reshape(1, D)
    b2r = b2.reshape(1, D)

    deg_k = _build_deg_kernel()
    agg_k = _build_agg_kernel()

    deg2 = deg_k(dst3, ones_ch, zeros_deg)
    d0 = deg2[:NN].reshape(NN, 1)
    d1 = deg2[DEGP:DEGP + NN].reshape(NN, 1)

    g1 = _dense1(x, W1, d0, d1)
    a1 = agg_k(g1, src, dst, zeros_agg)
    g2 = _dense2(a1, g1, d0, d1, W2, b1r)
    a2 = agg_k(g2, src, dst, zeros_agg)
    return _final(a2, g2, d0, d1, b2r, batch2d)


# speedup vs baseline: 27.0693x; 1.0524x over previous
import functools

import jax
import jax.numpy as jnp
from jax import lax
from jax.experimental import pallas as pl
from jax.experimental.pallas import tpu as pltpu
from jax.experimental.pallas import tpu_sc as plsc

NN = 10000
NE = 320000
D = 128
NG = 64

NC = 2
NS = 16
NW = NC * NS
CH = 80
EPT = NE // NW
NCH = EPT // CH
CHP = 128
EPTP = 10240
NCHP = EPTP // CHP
PAD_PT = EPTP - EPT
NNP = 10240
ROWS_PT = NNP // NS
DRAIN = CHP
DEGP = 10240
DEG_PT = DEGP // NS


@functools.lru_cache(maxsize=None)
def _build_deg_kernel():
    mesh = plsc.VectorSubcoreMesh(core_axis_name="c", subcore_axis_name="s")

    @functools.partial(
        pl.kernel,
        mesh=mesh,
        out_type=jax.ShapeDtypeStruct((NC * DEGP,), jnp.float32),
        scratch_types=[
            pltpu.VMEM((NCH, CH), jnp.int32),
            pltpu.VMEM((CH,), jnp.float32),
            pltpu.VMEM((DEG_PT,), jnp.float32),
            pltpu.VMEM_SHARED((DEGP,), jnp.float32),
        ],
    )
    def _deg_kernel(dst_hbm, ones_hbm, zeros_hbm, out_hbm, dst_v, ones_v, buf_v, acc_sh):
        c = lax.axis_index("c")
        s = lax.axis_index("s")
        wid = c * NS + s
        pltpu.sync_copy(zeros_hbm, buf_v)
        pltpu.sync_copy(buf_v, acc_sh.at[pl.ds(s * DEG_PT, DEG_PT)])
        pltpu.sync_copy(ones_hbm, ones_v)
        pltpu.sync_copy(dst_hbm.at[wid], dst_v)
        plsc.subcore_barrier()

        def body(i, carry):
            pltpu.sync_copy(ones_v, acc_sh.at[dst_v.at[i]], add=True)
            return carry

        lax.fori_loop(0, NCH, body, 0)
        plsc.subcore_barrier()

        pltpu.sync_copy(acc_sh.at[pl.ds(s * DEG_PT, DEG_PT)], buf_v)
        pltpu.sync_copy(buf_v, out_hbm.at[pl.ds(c * DEGP + s * DEG_PT, DEG_PT)])

    return _deg_kernel


@functools.lru_cache(maxsize=None)
def _build_agg_kernel():
    mesh = plsc.VectorSubcoreMesh(core_axis_name="c", subcore_axis_name="s")

    @functools.partial(
        pl.kernel,
        mesh=mesh,
        out_type=jax.ShapeDtypeStruct((NC * NNP, D), jnp.float32),
        scratch_types=[
            pltpu.VMEM((CHP,), jnp.int32),
            pltpu.VMEM((CHP,), jnp.int32),
            pltpu.VMEM((CHP,), jnp.int32),
            pltpu.VMEM((CHP,), jnp.int32),
            pltpu.VMEM((CHP, D), jnp.float32),
            pltpu.VMEM((CHP, D), jnp.float32),
            pltpu.VMEM_SHARED((NNP, D), jnp.float32),
            pltpu.SemaphoreType.DMA,
            pltpu.SemaphoreType.DMA,
            pltpu.SemaphoreType.DMA,
            pltpu.SemaphoreType.DMA,
        ],
    )
    def _agg_kernel(g_hbm, src_hbm, dst_hbm, zeros_hbm, out_hbm,
                    s0_v, d0_v, s1_v, d1_v, r0_v, r1_v, acc_sh,
                    isem0, isem1, gsem0, gsem1):
        c = lax.axis_index("c")
        s = lax.axis_index("s")
        eb = (c * NS + s) * EPTP

        def idx_start(ch, sv, dv, sem):
            pltpu.async_copy(src_hbm.at[pl.ds(eb + ch * CHP, CHP)], sv, sem)
            pltpu.async_copy(dst_hbm.at[pl.ds(eb + ch * CHP, CHP)], dv, sem)

        def idx_wait(ch, sv, dv, sem):
            pltpu.make_async_copy(src_hbm.at[pl.ds(eb + ch * CHP, CHP)], sv, sem).wait()
            pltpu.make_async_copy(dst_hbm.at[pl.ds(eb + ch * CHP, CHP)], dv, sem).wait()

        def gather_start(sv, rv, sem):
            pltpu.async_copy(g_hbm.at[sv], rv, sem)

        def gather_wait(sv, rv, sem):
            pltpu.make_async_copy(g_hbm.at[sv], rv, sem).wait()

        pltpu.sync_copy(zeros_hbm, r0_v)
        for k in range(ROWS_PT // DRAIN):
            pltpu.async_copy(r0_v, acc_sh.at[pl.ds(s * ROWS_PT + k * DRAIN, DRAIN)], gsem0)
        for k in range(ROWS_PT // DRAIN):
            pltpu.make_async_copy(
                r0_v, acc_sh.at[pl.ds(s * ROWS_PT + k * DRAIN, DRAIN)], gsem0).wait()
        plsc.subcore_barrier()

        idx_start(0, s0_v, d0_v, isem0)
        idx_start(1, s1_v, d1_v, isem1)
        idx_wait(0, s0_v, d0_v, isem0)
        gather_start(s0_v, r0_v, gsem0)

        def body(i, carry):
            a = 2 * i
            idx_wait(a + 1, s1_v, d1_v, isem1)
            gather_start(s1_v, r1_v, gsem1)
            gather_wait(s0_v, r0_v, gsem0)
            pltpu.sync_copy(r0_v, acc_sh.at[d0_v], add=True)
            ne = jnp.minimum(a + 2, NCHP - 1)
            no = jnp.minimum(a + 3, NCHP - 1)
            idx_start(ne, s0_v, d0_v, isem0)
            gather_wait(s1_v, r1_v, gsem1)
            pltpu.sync_copy(r1_v, acc_sh.at[d1_v], add=True)
            idx_start(no, s1_v, d1_v, isem1)
            idx_wait(ne, s0_v, d0_v, isem0)
            gather_start(s0_v, r0_v, gsem0)
            return carry

        lax.fori_loop(0, NCHP // 2, body, 0)
        gather_wait(s0_v, r0_v, gsem0)
        idx_wait(NCHP - 1, s1_v, d1_v, isem1)
        plsc.subcore_barrier()

        for k in range(ROWS_PT // DRAIN):
            r = s * ROWS_PT + k * DRAIN
            rb = r0_v if k % 2 == 0 else r1_v
            sm = gsem0 if k % 2 == 0 else gsem1
            if k >= 2:
                rp = s * ROWS_PT + (k - 2) * DRAIN
                pltpu.make_async_copy(rb, out_hbm.at[pl.ds(c * NNP + rp, DRAIN)], sm).wait()
            pltpu.sync_copy(acc_sh.at[pl.ds(r, DRAIN)], rb)
            pltpu.async_copy(rb, out_hbm.at[pl.ds(c * NNP + r, DRAIN)], sm)
        for k in range(ROWS_PT // DRAIN - 2, ROWS_PT // DRAIN):
            r = s * ROWS_PT + k * DRAIN
            rb = r0_v if k % 2 == 0 else r1_v
            sm = gsem0 if k % 2 == 0 else gsem1
            pltpu.make_async_copy(rb, out_hbm.at[pl.ds(c * NNP + r, DRAIN)], sm).wait()

    return _agg_kernel


def _dense1_body(x_ref, w_ref, d0_ref, d1_ref, g_ref):
    dinv = lax.rsqrt(d0_ref[...] + d1_ref[...] + 1.0)
    h = jnp.dot(x_ref[...], w_ref[...], preferred_element_type=jnp.float32)
    g_ref[...] = h * dinv


def _dense2_body(agg_ref, g1_ref, d0_ref, d1_ref, w_ref, b_ref, g2_ref):
    dinv = lax.rsqrt(d0_ref[...] + d1_ref[...] + 1.0)
    a = agg_ref[pl.ds(0, NN), :] + agg_ref[pl.ds(NNP, NN), :] + g1_ref[...]
    out1 = jnp.maximum(a * dinv + b_ref[...], 0.0)
    h2 = jnp.dot(out1, w_ref[...], preferred_element_type=jnp.float32)
    g2_ref[...] = h2 * dinv


def _final_body(agg_ref, g2_ref, d0_ref, d1_ref, b_ref, batch_ref, out_ref):
    dinv = lax.rsqrt(d0_ref[...] + d1_ref[...] + 1.0)
    a = agg_ref[pl.ds(0, NN), :] + agg_ref[pl.ds(NNP, NN), :] + g2_ref[...]
    out2 = jnp.maximum(a * dinv + b_ref[...], 0.0)
    gids = lax.broadcasted_iota(jnp.int32, (NG, NN), 0)
    p = (gids == batch_ref[...]).astype(jnp.float32)
    sums = jnp.dot(p, out2, preferred_element_type=jnp.float32)
    counts = jnp.sum(p, axis=1, keepdims=True)
    out_ref[...] = sums / jnp.maximum(counts, 1.0)


_dense1 = pl.pallas_call(
    _dense1_body, out_shape=jax.ShapeDtypeStruct((NN, D), jnp.float32))
_dense2 = pl.pallas_call(
    _dense2_body, out_shape=jax.ShapeDtypeStruct((NN, D), jnp.float32))
_final = pl.pallas_call(
    _final_body, out_shape=jax.ShapeDtypeStruct((NG, D), jnp.float32))


def kernel(x, edge_index, batch, W1, b1, W2, b2):
    src = edge_index[0].astype(jnp.int32)
    dst = edge_index[1].astype(jnp.int32)
    dst3 = dst.reshape(NW, NCH, CH)
    pad_iota = jnp.arange(NW * PAD_PT, dtype=jnp.int32)
    pad_src = ((pad_iota * 41) % NN).reshape(NW, PAD_PT)
    pad_dst = (NN + (pad_iota % (NNP - NN))).reshape(NW, PAD_PT)
    srcp = jnp.concatenate([src.reshape(NW, EPT), pad_src], axis=1).reshape(-1)
    dstp = jnp.concatenate([dst.reshape(NW, EPT), pad_dst], axis=1).reshape(-1)
    batch2d = batch.astype(jnp.int32).reshape(1, NN)
    ones_ch = jnp.ones((CH,), jnp.float32)
    zeros_deg = jnp.zeros((DEG_PT,), jnp.float32)
    zeros_agg = jnp.zeros((CHP, D), jnp.float32)
    b1r = b1.reshape(1, D)
    b2r = b2.reshape(1, D)

    deg_k = _build_deg_kernel()
    agg_k = _build_agg_kernel()

    deg2 = deg_k(dst3, ones_ch, zeros_deg)
    d0 = deg2[:NN].reshape(NN, 1)
    d1 = deg2[DEGP:DEGP + NN].reshape(NN, 1)

    g1 = _dense1(x, W1, d0, d1)
    a1 = agg_k(g1, srcp, dstp, zeros_agg)
    g2 = _dense2(a1, g1, d0, d1, W2, b1r)
    a2 = agg_k(g2, srcp, dstp, zeros_agg)
    return _final(a2, g2, d0, d1, b2r, batch2d)


# speedup vs baseline: 27.4901x; 1.0155x over previous
import functools

import jax
import jax.numpy as jnp
from jax import lax
from jax.experimental import pallas as pl
from jax.experimental.pallas import tpu as pltpu
from jax.experimental.pallas import tpu_sc as plsc

NN = 10000
NE = 320000
D = 128
NG = 64

NC = 2
NS = 16
NW = NC * NS
CH = 80
EPT = NE // NW
NCH = EPT // CH
CHP = 128
EPTP = 10240
NCHP = EPTP // CHP
PAD_PT = EPTP - EPT
NNP = 10240
ROWS_PT = NNP // NS
DRAIN = CHP
DEGP = 10240
DEG_PT = DEGP // NS


@functools.lru_cache(maxsize=None)
def _build_deg_kernel():
    mesh = plsc.VectorSubcoreMesh(core_axis_name="c", subcore_axis_name="s")

    @functools.partial(
        pl.kernel,
        mesh=mesh,
        out_type=jax.ShapeDtypeStruct((NC * DEGP,), jnp.float32),
        scratch_types=[
            pltpu.VMEM((NCH, CH), jnp.int32),
            pltpu.VMEM((CH,), jnp.float32),
            pltpu.VMEM((DEG_PT,), jnp.float32),
            pltpu.VMEM_SHARED((DEGP,), jnp.float32),
        ],
    )
    def _deg_kernel(dst_hbm, ones_hbm, zeros_hbm, out_hbm, dst_v, ones_v, buf_v, acc_sh):
        c = lax.axis_index("c")
        s = lax.axis_index("s")
        wid = c * NS + s
        pltpu.sync_copy(zeros_hbm, buf_v)
        pltpu.sync_copy(buf_v, acc_sh.at[pl.ds(s * DEG_PT, DEG_PT)])
        pltpu.sync_copy(ones_hbm, ones_v)
        pltpu.sync_copy(dst_hbm.at[wid], dst_v)
        plsc.subcore_barrier()

        def body(i, carry):
            pltpu.sync_copy(ones_v, acc_sh.at[dst_v.at[i]], add=True)
            return carry

        lax.fori_loop(0, NCH, body, 0)
        plsc.subcore_barrier()

        pltpu.sync_copy(acc_sh.at[pl.ds(s * DEG_PT, DEG_PT)], buf_v)
        pltpu.sync_copy(buf_v, out_hbm.at[pl.ds(c * DEGP + s * DEG_PT, DEG_PT)])

    return _deg_kernel


@functools.lru_cache(maxsize=None)
def _build_agg_kernel():
    mesh = plsc.VectorSubcoreMesh(core_axis_name="c", subcore_axis_name="s")

    @functools.partial(
        pl.kernel,
        mesh=mesh,
        out_type=jax.ShapeDtypeStruct((NC * NNP, D), jnp.float32),
        scratch_types=[
            pltpu.VMEM((CHP,), jnp.int32),
            pltpu.VMEM((CHP,), jnp.int32),
            pltpu.VMEM((CHP,), jnp.int32),
            pltpu.VMEM((CHP,), jnp.int32),
            pltpu.VMEM((CHP, D), jnp.float32),
            pltpu.VMEM((CHP, D), jnp.float32),
            pltpu.VMEM_SHARED((NNP, D), jnp.float32),
            pltpu.SemaphoreType.DMA,
            pltpu.SemaphoreType.DMA,
            pltpu.SemaphoreType.DMA,
            pltpu.SemaphoreType.DMA,
            pltpu.SemaphoreType.DMA,
            pltpu.SemaphoreType.DMA,
        ],
    )
    def _agg_kernel(g_hbm, src_hbm, dst_hbm, zeros_hbm, out_hbm,
                    s0_v, d0_v, s1_v, d1_v, r0_v, r1_v, acc_sh,
                    isem0, isem1, gsem0, gsem1, ssem0, ssem1):
        c = lax.axis_index("c")
        s = lax.axis_index("s")
        eb = (c * NS + s) * EPTP

        def idx_start(ch, sv, dv, sem):
            pltpu.async_copy(src_hbm.at[pl.ds(eb + ch * CHP, CHP)], sv, sem)
            pltpu.async_copy(dst_hbm.at[pl.ds(eb + ch * CHP, CHP)], dv, sem)

        def idx_wait(ch, sv, dv, sem):
            pltpu.make_async_copy(src_hbm.at[pl.ds(eb + ch * CHP, CHP)], sv, sem).wait()
            pltpu.make_async_copy(dst_hbm.at[pl.ds(eb + ch * CHP, CHP)], dv, sem).wait()

        def gather_start(sv, rv, sem):
            pltpu.async_copy(g_hbm.at[sv], rv, sem)

        def gather_wait(sv, rv, sem):
            pltpu.make_async_copy(g_hbm.at[sv], rv, sem).wait()

        pltpu.sync_copy(zeros_hbm, r0_v)
        for k in range(ROWS_PT // DRAIN):
            pltpu.async_copy(r0_v, acc_sh.at[pl.ds(s * ROWS_PT + k * DRAIN, DRAIN)], gsem0)
        for k in range(ROWS_PT // DRAIN):
            pltpu.make_async_copy(
                r0_v, acc_sh.at[pl.ds(s * ROWS_PT + k * DRAIN, DRAIN)], gsem0).wait()
        plsc.subcore_barrier()

        idx_start(0, s0_v, d0_v, isem0)
        idx_start(1, s1_v, d1_v, isem1)
        idx_wait(0, s0_v, d0_v, isem0)
        gather_start(s0_v, r0_v, gsem0)

        def body(i, carry):
            a = 2 * i
            idx_wait(a + 1, s1_v, d1_v, isem1)
            gather_start(s1_v, r1_v, gsem1)
            gather_wait(s0_v, r0_v, gsem0)
            pltpu.async_copy(r0_v, acc_sh.at[d0_v], ssem0, add=True)
            gather_wait(s1_v, r1_v, gsem1)
            pltpu.async_copy(r1_v, acc_sh.at[d1_v], ssem1, add=True)
            ne = jnp.minimum(a + 2, NCHP - 1)
            no = jnp.minimum(a + 3, NCHP - 1)
            pltpu.make_async_copy(r0_v, acc_sh.at[d0_v], ssem0).wait()
            idx_start(ne, s0_v, d0_v, isem0)
            idx_wait(ne, s0_v, d0_v, isem0)
            gather_start(s0_v, r0_v, gsem0)
            pltpu.make_async_copy(r1_v, acc_sh.at[d1_v], ssem1).wait()
            idx_start(no, s1_v, d1_v, isem1)
            return carry

        lax.fori_loop(0, NCHP // 2, body, 0)
        gather_wait(s0_v, r0_v, gsem0)
        idx_wait(NCHP - 1, s1_v, d1_v, isem1)
        plsc.subcore_barrier()

        for k in range(ROWS_PT // DRAIN):
            r = s * ROWS_PT + k * DRAIN
            rb = r0_v if k % 2 == 0 else r1_v
            sm = gsem0 if k % 2 == 0 else gsem1
            if k >= 2:
                rp = s * ROWS_PT + (k - 2) * DRAIN
                pltpu.make_async_copy(rb, out_hbm.at[pl.ds(c * NNP + rp, DRAIN)], sm).wait()
            pltpu.sync_copy(acc_sh.at[pl.ds(r, DRAIN)], rb)
            pltpu.async_copy(rb, out_hbm.at[pl.ds(c * NNP + r, DRAIN)], sm)
        for k in range(ROWS_PT // DRAIN - 2, ROWS_PT // DRAIN):
            r = s * ROWS_PT + k * DRAIN
            rb = r0_v if k % 2 == 0 else r1_v
            sm = gsem0 if k % 2 == 0 else gsem1
            pltpu.make_async_copy(rb, out_hbm.at[pl.ds(c * NNP + r, DRAIN)], sm).wait()

    return _agg_kernel


def _dense1_body(x_ref, w_ref, d0_ref, d1_ref, g_ref):
    dinv = lax.rsqrt(d0_ref[...] + d1_ref[...] + 1.0)
    h = jnp.dot(x_ref[...], w_ref[...], preferred_element_type=jnp.float32)
    g_ref[...] = h * dinv


def _dense2_body(agg_ref, g1_ref, d0_ref, d1_ref, w_ref, b_ref, g2_ref):
    dinv = lax.rsqrt(d0_ref[...] + d1_ref[...] + 1.0)
    a = agg_ref[pl.ds(0, NN), :] + agg_ref[pl.ds(NNP, NN), :] + g1_ref[...]
    out1 = jnp.maximum(a * dinv + b_ref[...], 0.0)
    h2 = jnp.dot(out1, w_ref[...], preferred_element_type=jnp.float32)
    g2_ref[...] = h2 * dinv


def _final_body(agg_ref, g2_ref, d0_ref, d1_ref, b_ref, batch_ref, out_ref):
    dinv = lax.rsqrt(d0_ref[...] + d1_ref[...] + 1.0)
    a = agg_ref[pl.ds(0, NN), :] + agg_ref[pl.ds(NNP, NN), :] + g2_ref[...]
    out2 = jnp.maximum(a * dinv + b_ref[...], 0.0)
    gids = lax.broadcasted_iota(jnp.int32, (NG, NN), 0)
    p = (gids == batch_ref[...]).astype(jnp.float32)
    sums = jnp.dot(p, out2, preferred_element_type=jnp.float32)
    counts = jnp.sum(p, axis=1, keepdims=True)
    out_ref[...] = sums / jnp.maximum(counts, 1.0)


_dense1 = pl.pallas_call(
    _dense1_body, out_shape=jax.ShapeDtypeStruct((NN, D), jnp.float32))
_dense2 = pl.pallas_call(
    _dense2_body, out_shape=jax.ShapeDtypeStruct((NN, D), jnp.float32))
_final = pl.pallas_call(
    _final_body, out_shape=jax.ShapeDtypeStruct((NG, D), jnp.float32))


def kernel(x, edge_index, batch, W1, b1, W2, b2):
    src = edge_index[0].astype(jnp.int32)
    dst = edge_index[1].astype(jnp.int32)
    dst3 = dst.reshape(NW, NCH, CH)
    pad_iota = jnp.arange(NW * PAD_PT, dtype=jnp.int32)
    pad_src = ((pad_iota * 41) % NN).reshape(NW, PAD_PT)
    pad_dst = (NN + (pad_iota % (NNP - NN))).reshape(NW, PAD_PT)
    srcp = jnp.concatenate([src.reshape(NW, EPT), pad_src], axis=1).reshape(-1)
    dstp = jnp.concatenate([dst.reshape(NW, EPT), pad_dst], axis=1).reshape(-1)
    batch2d = batch.astype(jnp.int32).reshape(1, NN)
    ones_ch = jnp.ones((CH,), jnp.float32)
    zeros_deg = jnp.zeros((DEG_PT,), jnp.float32)
    zeros_agg = jnp.zeros((CHP, D), jnp.float32)
    b1r = b1.reshape(1, D)
    b2r = b2.reshape(1, D)

    deg_k = _build_deg_kernel()
    agg_k = _build_agg_kernel()

    deg2 = deg_k(dst3, ones_ch, zeros_deg)
    d0 = deg2[:NN].reshape(NN, 1)
    d1 = deg2[DEGP:DEGP + NN].reshape(NN, 1)

    g1 = _dense1(x, W1, d0, d1)
    a1 = agg_k(g1, srcp, dstp, zeros_agg)
    g2 = _dense2(a1, g1, d0, d1, W2, b1r)
    a2 = agg_k(g2, srcp, dstp, zeros_agg)
    return _final(a2, g2, d0, d1, b2r, batch2d)


# speedup vs baseline: 30.5994x; 1.1131x over previous
import functools

import jax
import jax.numpy as jnp
from jax import lax
from jax.experimental import pallas as pl
from jax.experimental.pallas import tpu as pltpu
from jax.experimental.pallas import tpu_sc as plsc

NN = 10000
NE = 320000
D = 128
NG = 64

NC = 2
NS = 16
NW = NC * NS
CH = 80
EPT = NE // NW
NCH = EPT // CH
CHP = 88
EPTP = 10032
NCHP = EPTP // CHP
PAD_PT = EPTP - EPT
NNP = 10240
ROWS_PT = NNP // NS
DRAIN = 80
DEGP = 10240
DEG_PT = DEGP // NS


@functools.lru_cache(maxsize=None)
def _build_deg_kernel():
    mesh = plsc.VectorSubcoreMesh(core_axis_name="c", subcore_axis_name="s")

    @functools.partial(
        pl.kernel,
        mesh=mesh,
        out_type=jax.ShapeDtypeStruct((NC * DEGP,), jnp.float32),
        scratch_types=[
            pltpu.VMEM((NCH, CH), jnp.int32),
            pltpu.VMEM((CH,), jnp.float32),
            pltpu.VMEM((DEG_PT,), jnp.float32),
            pltpu.VMEM_SHARED((DEGP,), jnp.float32),
        ],
    )
    def _deg_kernel(dst_hbm, ones_hbm, zeros_hbm, out_hbm, dst_v, ones_v, buf_v, acc_sh):
        c = lax.axis_index("c")
        s = lax.axis_index("s")
        wid = c * NS + s
        pltpu.sync_copy(zeros_hbm, buf_v)
        pltpu.sync_copy(buf_v, acc_sh.at[pl.ds(s * DEG_PT, DEG_PT)])
        pltpu.sync_copy(ones_hbm, ones_v)
        pltpu.sync_copy(dst_hbm.at[wid], dst_v)
        plsc.subcore_barrier()

        def body(i, carry):
            pltpu.sync_copy(ones_v, acc_sh.at[dst_v.at[i]], add=True)
            return carry

        lax.fori_loop(0, NCH, body, 0)
        plsc.subcore_barrier()

        pltpu.sync_copy(acc_sh.at[pl.ds(s * DEG_PT, DEG_PT)], buf_v)
        pltpu.sync_copy(buf_v, out_hbm.at[pl.ds(c * DEGP + s * DEG_PT, DEG_PT)])

    return _deg_kernel


@functools.lru_cache(maxsize=None)
def _build_agg_kernel():
    mesh = plsc.VectorSubcoreMesh(core_axis_name="c", subcore_axis_name="s")

    @functools.partial(
        pl.kernel,
        mesh=mesh,
        out_type=jax.ShapeDtypeStruct((NC * NNP, D), jnp.float32),
        scratch_types=[
            pltpu.VMEM((CHP,), jnp.int32),
            pltpu.VMEM((CHP,), jnp.int32),
            pltpu.VMEM((CHP,), jnp.int32),
            pltpu.VMEM((CHP,), jnp.int32),
            pltpu.VMEM((CHP,), jnp.int32),
            pltpu.VMEM((CHP,), jnp.int32),
            pltpu.VMEM((CHP, D), jnp.float32),
            pltpu.VMEM((CHP, D), jnp.float32),
            pltpu.VMEM((CHP, D), jnp.float32),
            pltpu.VMEM_SHARED((NNP, D), jnp.float32),
            pltpu.SemaphoreType.DMA,
            pltpu.SemaphoreType.DMA,
            pltpu.SemaphoreType.DMA,
            pltpu.SemaphoreType.DMA,
            pltpu.SemaphoreType.DMA,
            pltpu.SemaphoreType.DMA,
            pltpu.SemaphoreType.DMA,
            pltpu.SemaphoreType.DMA,
            pltpu.SemaphoreType.DMA,
        ],
    )
    def _agg_kernel(g_hbm, src_hbm, dst_hbm, zeros_hbm, out_hbm,
                    s0_v, d0_v, s1_v, d1_v, s2_v, d2_v, r0_v, r1_v, r2_v, acc_sh,
                    isem0, isem1, isem2, gsem0, gsem1, gsem2, ssem0, ssem1, ssem2):
        c = lax.axis_index("c")
        s = lax.axis_index("s")
        eb = (c * NS + s) * EPTP
        sv = (s0_v, s1_v, s2_v)
        dv = (d0_v, d1_v, d2_v)
        rv = (r0_v, r1_v, r2_v)
        isem = (isem0, isem1, isem2)
        gsem = (gsem0, gsem1, gsem2)
        ssem = (ssem0, ssem1, ssem2)

        def idx_start(ch, k):
            pltpu.async_copy(src_hbm.at[pl.ds(eb + ch * CHP, CHP)], sv[k], isem[k])
            pltpu.async_copy(dst_hbm.at[pl.ds(eb + ch * CHP, CHP)], dv[k], isem[k])

        def idx_wait(ch, k):
            pltpu.make_async_copy(src_hbm.at[pl.ds(eb + ch * CHP, CHP)], sv[k], isem[k]).wait()
            pltpu.make_async_copy(dst_hbm.at[pl.ds(eb + ch * CHP, CHP)], dv[k], isem[k]).wait()

        def gather_start(k):
            pltpu.async_copy(g_hbm.at[sv[k]], rv[k], gsem[k])

        def gather_wait(k):
            pltpu.make_async_copy(g_hbm.at[sv[k]], rv[k], gsem[k]).wait()

        def scatter_start(k):
            pltpu.async_copy(rv[k], acc_sh.at[dv[k]], ssem[k], add=True)

        def scatter_wait(k):
            pltpu.make_async_copy(rv[k], acc_sh.at[dv[k]], ssem[k]).wait()

        zb = r0_v.at[pl.ds(0, DRAIN)]
        pltpu.sync_copy(zeros_hbm, zb)
        for k in range(ROWS_PT // DRAIN):
            pltpu.async_copy(zb, acc_sh.at[pl.ds(s * ROWS_PT + k * DRAIN, DRAIN)], gsem0)
        for k in range(ROWS_PT // DRAIN):
            pltpu.make_async_copy(
                zb, acc_sh.at[pl.ds(s * ROWS_PT + k * DRAIN, DRAIN)], gsem0).wait()
        plsc.subcore_barrier()

        def step(ch, a, first=False):
            cc = (a + 2) % 3
            gather_wait(a)
            scatter_start(a)
            if not first:
                scatter_wait(cc)
            nxt = jnp.minimum(ch + 2, NCHP - 1)
            idx_start(nxt, cc)
            idx_wait(nxt, cc)
            gather_start(cc)

        idx_start(0, 0)
        idx_start(1, 1)
        idx_wait(0, 0)
        gather_start(0)
        idx_wait(1, 1)
        gather_start(1)
        step(0, 0, first=True)
        step(1, 1)
        step(2, 2)

        def body(i, carry):
            ch = 3 * i
            step(ch, 0)
            step(ch + 1, 1)
            step(ch + 2, 2)
            return carry

        lax.fori_loop(1, NCHP // 3, body, 0)
        scatter_wait(2)
        gather_wait(0)
        gather_wait(1)
        plsc.subcore_barrier()

        b0 = r0_v.at[pl.ds(0, DRAIN)]
        b1 = r1_v.at[pl.ds(0, DRAIN)]
        for k in range(ROWS_PT // DRAIN):
            r = s * ROWS_PT + k * DRAIN
            rb = b0 if k % 2 == 0 else b1
            sm = gsem0 if k % 2 == 0 else gsem1
            if k >= 2:
                rp = s * ROWS_PT + (k - 2) * DRAIN
                pltpu.make_async_copy(rb, out_hbm.at[pl.ds(c * NNP + rp, DRAIN)], sm).wait()
            pltpu.sync_copy(acc_sh.at[pl.ds(r, DRAIN)], rb)
            pltpu.async_copy(rb, out_hbm.at[pl.ds(c * NNP + r, DRAIN)], sm)
        for k in range(ROWS_PT // DRAIN - 2, ROWS_PT // DRAIN):
            r = s * ROWS_PT + k * DRAIN
            rb = b0 if k % 2 == 0 else b1
            sm = gsem0 if k % 2 == 0 else gsem1
            pltpu.make_async_copy(rb, out_hbm.at[pl.ds(c * NNP + r, DRAIN)], sm).wait()

    return _agg_kernel


def _dense1_body(x_ref, w_ref, d0_ref, d1_ref, g_ref):
    dinv = lax.rsqrt(d0_ref[...] + d1_ref[...] + 1.0)
    h = jnp.dot(x_ref[...], w_ref[...], preferred_element_type=jnp.float32)
    g_ref[...] = h * dinv


def _dense2_body(agg_ref, g1_ref, d0_ref, d1_ref, w_ref, b_ref, g2_ref):
    dinv = lax.rsqrt(d0_ref[...] + d1_ref[...] + 1.0)
    a = agg_ref[pl.ds(0, NN), :] + agg_ref[pl.ds(NNP, NN), :] + g1_ref[...]
    out1 = jnp.maximum(a * dinv + b_ref[...], 0.0)
    h2 = jnp.dot(out1, w_ref[...], preferred_element_type=jnp.float32)
    g2_ref[...] = h2 * dinv


def _final_body(agg_ref, g2_ref, d0_ref, d1_ref, b_ref, batch_ref, out_ref):
    dinv = lax.rsqrt(d0_ref[...] + d1_ref[...] + 1.0)
    a = agg_ref[pl.ds(0, NN), :] + agg_ref[pl.ds(NNP, NN), :] + g2_ref[...]
    out2 = jnp.maximum(a * dinv + b_ref[...], 0.0)
    gids = lax.broadcasted_iota(jnp.int32, (NG, NN), 0)
    p = (gids == batch_ref[...]).astype(jnp.float32)
    sums = jnp.dot(p, out2, preferred_element_type=jnp.float32)
    counts = jnp.sum(p, axis=1, keepdims=True)
    out_ref[...] = sums / jnp.maximum(counts, 1.0)


_dense1 = pl.pallas_call(
    _dense1_body, out_shape=jax.ShapeDtypeStruct((NN, D), jnp.float32))
_dense2 = pl.pallas_call(
    _dense2_body, out_shape=jax.ShapeDtypeStruct((NN, D), jnp.float32))
_final = pl.pallas_call(
    _final_body, out_shape=jax.ShapeDtypeStruct((NG, D), jnp.float32))


def kernel(x, edge_index, batch, W1, b1, W2, b2):
    src = edge_index[0].astype(jnp.int32)
    dst = edge_index[1].astype(jnp.int32)
    dst3 = dst.reshape(NW, NCH, CH)
    pad_iota = jnp.arange(NW * PAD_PT, dtype=jnp.int32)
    pad_src = ((pad_iota * 41) % NN).reshape(NW, PAD_PT)
    pad_dst = (NN + (pad_iota % (NNP - NN))).reshape(NW, PAD_PT)
    srcp = jnp.concatenate([src.reshape(NW, EPT), pad_src], axis=1).reshape(-1)
    dstp = jnp.concatenate([dst.reshape(NW, EPT), pad_dst], axis=1).reshape(-1)
    batch2d = batch.astype(jnp.int32).reshape(1, NN)
    ones_ch = jnp.ones((CH,), jnp.float32)
    zeros_deg = jnp.zeros((DEG_PT,), jnp.float32)
    zeros_agg = jnp.zeros((DRAIN, D), jnp.float32)
    b1r = b1.reshape(1, D)
    b2r = b2.reshape(1, D)

    deg_k = _build_deg_kernel()
    agg_k = _build_agg_kernel()

    deg2 = deg_k(dst3, ones_ch, zeros_deg)
    d0 = deg2[:NN].reshape(NN, 1)
    d1 = deg2[DEGP:DEGP + NN].reshape(NN, 1)

    g1 = _dense1(x, W1, d0, d1)
    a1 = agg_k(g1, srcp, dstp, zeros_agg)
    g2 = _dense2(a1, g1, d0, d1, W2, b1r)
    a2 = agg_k(g2, srcp, dstp, zeros_agg)
    return _final(a2, g2, d0, d1, b2r, batch2d)
